# baseline (speedup 1.0000x reference)
import functools

import jax
import jax.numpy as jnp
from jax import lax
from jax.experimental import pallas as pl
from jax.experimental.pallas import tpu as pltpu
from jax.experimental.pallas import tpu_sc as plsc

NI, NJ = 16384, 100
V, D = 256, 64
NC, NS, L = 2, 16, 16
NW = NC * NS
IPW = NI // NW
NIB = IPW // 128
NKB = D // 8
NTILES = NKB * NIB
NSEG = IPW // L

_mesh = plsc.VectorSubcoreMesh(
    core_axis_name="c", subcore_axis_name="s", num_cores=NC, num_subcores=NS
)


@functools.partial(
    pl.kernel,
    out_type=jax.ShapeDtypeStruct((NJ, D, NI), jnp.float32),
    mesh=_mesh,
    compiler_params=pltpu.CompilerParams(needs_layout_passes=False),
    scratch_types=[
        pltpu.VMEM((D * V,), jnp.float32),
        pltpu.VMEM((IPW,), jnp.int32),
        pltpu.VMEM((IPW,), jnp.int32),
        pltpu.VMEM((NTILES, 8, 128), jnp.float32),
        pltpu.VMEM((NTILES, 8, 128), jnp.float32),
        pltpu.SemaphoreType.DMA,
        pltpu.SemaphoreType.DMA,
        pltpu.SemaphoreType.DMA,
        pltpu.SemaphoreType.DMA,
    ],
)
def _sc_embed(wt_hbm, idxt_hbm, out_hbm, table_v, idx0, idx1, st0, st1,
              isem0, isem1, osem0, osem1):
    wid = lax.axis_index("s") * NC + lax.axis_index("c")
    i0 = wid * IPW

    pltpu.sync_copy(wt_hbm, table_v)

    idx_bufs = (idx0, idx1)
    st_bufs = (st0, st1)
    isems = (isem0, isem1)
    osems = (osem0, osem1)

    for b in range(2):
        pltpu.async_copy(
            idxt_hbm.at[pl.ds(b * NI + i0, IPW)], idx_bufs[b], isems[b]
        )

    def compute_chunk(idx_st, st):
        @plsc.parallel_loop(0, NSEG, unroll=8)
        def seg(s):
            ib = lax.div(s, NKB)
            v = lax.rem(s, NKB)
            rowvec = idx_st[pl.ds(s * L, L)]
            for kp in range(D):
                vals = plsc.load_gather(table_v, [rowvec + kp * V])
                st[(kp // 8) * NIB + ib, kp % 8, pl.ds(v * L, L)] = vals

    def fire_out(st, j, sem):
        @plsc.parallel_loop(0, NTILES)
        def fire(t):
            kb = lax.div(t, NIB)
            ib = lax.rem(t, NIB)
            pltpu.async_copy(
                st.at[t],
                out_hbm.at[j, pl.ds(kb * 8, 8), pl.ds(i0 + ib * 128, 128)],
                sem,
            )

    def drain_out(st, sem):
        @plsc.parallel_loop(0, NTILES)
        def drain(t):
            pltpu.make_async_copy(
                st.at[0], out_hbm.at[0, pl.ds(0, 8), pl.ds(0, 128)], sem
            ).wait()

    def super_step(ss, carry):
        for b in range(2):
            j = 2 * ss + b
            pltpu.make_async_copy(
                idxt_hbm.at[pl.ds(0, IPW)], idx_bufs[b], isems[b]
            ).wait()

            @pl.when(ss > 0)
            def _drain():
                drain_out(st_bufs[b], osems[b])

            compute_chunk(idx_bufs[b], st_bufs[b])
            fire_out(st_bufs[b], j, osems[b])

            @pl.when(j + 2 < NJ)
            def _prefetch():
                pltpu.async_copy(
                    idxt_hbm.at[pl.ds((j + 2) * NI + i0, IPW)],
                    idx_bufs[b],
                    isems[b],
                )
        return carry

    lax.fori_loop(0, NJ // 2, super_step, 0)

    for b in range(2):
        drain_out(st_bufs[b], osems[b])


def kernel(channel_indices, weight):
    idxt = channel_indices.T.reshape(-1).astype(jnp.int32)
    wt = weight.T.reshape(-1)
    out = _sc_embed(wt, idxt)
    return jnp.transpose(out, (2, 0, 1))


# speedup vs baseline: 1.1006x; 1.1006x over previous
import functools

import jax
import jax.numpy as jnp
from jax import lax
from jax.experimental import pallas as pl
from jax.experimental.pallas import tpu as pltpu
from jax.experimental.pallas import tpu_sc as plsc

NI, NJ = 16384, 100
V, D = 256, 64
NC, NS, L = 2, 16, 16
NW = NC * NS
IPW = NI // NW
NIB = IPW // 128
NKB = D // 8
NTILES = NKB * NIB
NSEG = IPW // L

_mesh = plsc.VectorSubcoreMesh(
    core_axis_name="c", subcore_axis_name="s", num_cores=NC, num_subcores=NS
)


@functools.partial(
    pl.kernel,
    out_type=jax.ShapeDtypeStruct((NJ, D, NI), jnp.float32),
    mesh=_mesh,
    compiler_params=pltpu.CompilerParams(needs_layout_passes=False),
    scratch_types=[
        pltpu.VMEM((D * V,), jnp.float32),
        pltpu.VMEM((IPW,), jnp.int32),
        pltpu.VMEM((IPW,), jnp.int32),
        pltpu.VMEM((NTILES, 8, 128), jnp.float32),
        pltpu.VMEM((NTILES, 8, 128), jnp.float32),
        pltpu.SemaphoreType.DMA,
        pltpu.SemaphoreType.DMA,
        pltpu.SemaphoreType.DMA,
        pltpu.SemaphoreType.DMA,
    ],
)
def _sc_embed(wt_hbm, idxt_hbm, out_hbm, table_v, idx0, idx1, st0, st1,
              isem0, isem1, osem0, osem1):
    wid = lax.axis_index("s") * NC + lax.axis_index("c")
    i0 = wid * IPW

    pltpu.sync_copy(wt_hbm, table_v)

    idx_bufs = (idx0, idx1)
    st_bufs = (st0, st1)
    isems = (isem0, isem1)
    osems = (osem0, osem1)

    for b in range(2):
        pltpu.async_copy(
            idxt_hbm.at[pl.ds(b * NI + i0, IPW)], idx_bufs[b], isems[b]
        )

    def compute_chunk(idx_st, st):
        @plsc.parallel_loop(0, NSEG, unroll=4)
        def seg(s):
            ib = lax.div(s, NKB)
            v = lax.rem(s, NKB)
            rowvec = idx_st[pl.ds(s * L, L)]
            for kp in range(D):
                vals = plsc.load_gather(table_v, [rowvec + kp * V])
                st[(kp // 8) * NIB + ib, kp % 8, pl.ds(v * L, L)] = vals

    def fire_out(st, j, sem):
        @plsc.parallel_loop(0, NTILES)
        def fire(t):
            kb = lax.div(t, NIB)
            ib = lax.rem(t, NIB)
            pltpu.async_copy(
                st.at[t],
                out_hbm.at[j, pl.ds(kb * 8, 8), pl.ds(i0 + ib * 128, 128)],
                sem,
            )

    def drain_out(st, sem):
        @plsc.parallel_loop(0, NTILES)
        def drain(t):
            pltpu.make_async_copy(
                st.at[0], out_hbm.at[0, pl.ds(0, 8), pl.ds(0, 128)], sem
            ).wait()

    def super_step(ss, carry):
        for b in range(2):
            j = 2 * ss + b
            pltpu.make_async_copy(
                idxt_hbm.at[pl.ds(0, IPW)], idx_bufs[b], isems[b]
            ).wait()

            @pl.when(ss > 0)
            def _drain():
                drain_out(st_bufs[b], osems[b])

            compute_chunk(idx_bufs[b], st_bufs[b])
            fire_out(st_bufs[b], j, osems[b])

            @pl.when(j + 2 < NJ)
            def _prefetch():
                pltpu.async_copy(
                    idxt_hbm.at[pl.ds((j + 2) * NI + i0, IPW)],
                    idx_bufs[b],
                    isems[b],
                )
        return carry

    lax.fori_loop(0, NJ // 2, super_step, 0)

    for b in range(2):
        drain_out(st_bufs[b], osems[b])


def kernel(channel_indices, weight):
    idxt = channel_indices.T.reshape(-1).astype(jnp.int32)
    wt = weight.T.reshape(-1)
    out = _sc_embed(wt, idxt)
    return jnp.transpose(out, (2, 0, 1))


# speedup vs baseline: 1.1675x; 1.0608x over previous
import functools

import jax
import jax.numpy as jnp
from jax import lax
from jax.experimental import pallas as pl
from jax.experimental.pallas import tpu as pltpu
from jax.experimental.pallas import tpu_sc as plsc

NI, NJ = 16384, 100
V, D = 256, 64
NC, NS, L = 2, 16, 16
NW = NC * NS
IPW = NI // NW
NIB = IPW // 128
NKB = D // 8
NTILES = NKB * NIB
NSEG = IPW // L

_mesh = plsc.VectorSubcoreMesh(
    core_axis_name="c", subcore_axis_name="s", num_cores=NC, num_subcores=NS
)


@functools.partial(
    pl.kernel,
    out_type=jax.ShapeDtypeStruct((NJ, D, NI), jnp.float32),
    mesh=_mesh,
    compiler_params=pltpu.CompilerParams(needs_layout_passes=False),
    scratch_types=[
        pltpu.VMEM((D * V,), jnp.float32),
        pltpu.VMEM((IPW,), jnp.int32),
        pltpu.VMEM((IPW,), jnp.int32),
        pltpu.VMEM((NKB, 8, IPW), jnp.float32),
        pltpu.VMEM((NKB, 8, IPW), jnp.float32),
        pltpu.SemaphoreType.DMA,
        pltpu.SemaphoreType.DMA,
        pltpu.SemaphoreType.DMA,
        pltpu.SemaphoreType.DMA,
    ],
)
def _sc_embed(wt_hbm, idxt_hbm, out_hbm, table_v, idx0, idx1, st0, st1,
              isem0, isem1, osem0, osem1):
    wid = lax.axis_index("s") * NC + lax.axis_index("c")
    i0 = wid * IPW

    pltpu.sync_copy(wt_hbm, table_v)

    idx_bufs = (idx0, idx1)
    st_bufs = (st0, st1)
    isems = (isem0, isem1)
    osems = (osem0, osem1)

    for b in range(2):
        pltpu.async_copy(
            idxt_hbm.at[pl.ds(b * NI + i0, IPW)], idx_bufs[b], isems[b]
        )

    def compute_chunk(idx_st, st):
        @plsc.parallel_loop(0, NSEG, unroll=4)
        def seg(s):
            ib = lax.div(s, NKB)
            v = lax.rem(s, NKB)
            rowvec = idx_st[pl.ds(s * L, L)]
            for kp in range(D):
                vals = plsc.load_gather(table_v, [rowvec + kp * V])
                st[kp // 8, kp % 8, pl.ds(s * L, L)] = vals

    def fire_out(st, j, sem):
        @plsc.parallel_loop(0, NKB)
        def fire(kb):
            pltpu.async_copy(
                st.at[kb],
                out_hbm.at[j, pl.ds(kb * 8, 8), pl.ds(i0, IPW)],
                sem,
            )

    def drain_out(st, sem):
        @plsc.parallel_loop(0, NKB)
        def drain(kb):
            pltpu.make_async_copy(
                st.at[0], out_hbm.at[0, pl.ds(0, 8), pl.ds(0, IPW)], sem
            ).wait()

    def super_step(ss, carry):
        for b in range(2):
            j = 2 * ss + b
            pltpu.make_async_copy(
                idxt_hbm.at[pl.ds(0, IPW)], idx_bufs[b], isems[b]
            ).wait()

            @pl.when(ss > 0)
            def _drain():
                drain_out(st_bufs[b], osems[b])

            compute_chunk(idx_bufs[b], st_bufs[b])
            fire_out(st_bufs[b], j, osems[b])

            @pl.when(j + 2 < NJ)
            def _prefetch():
                pltpu.async_copy(
                    idxt_hbm.at[pl.ds((j + 2) * NI + i0, IPW)],
                    idx_bufs[b],
                    isems[b],
                )
        return carry

    lax.fori_loop(0, NJ // 2, super_step, 0)

    for b in range(2):
        drain_out(st_bufs[b], osems[b])


def kernel(channel_indices, weight):
    idxt = channel_indices.T.reshape(-1).astype(jnp.int32)
    wt = weight.T.reshape(-1)
    out = _sc_embed(wt, idxt)
    return jnp.transpose(out, (2, 0, 1))
